# baseline (speedup 1.0000x reference)
import functools

import jax
import jax.numpy as jnp
from jax import lax
from jax.experimental import pallas as pl
from jax.experimental.pallas import tpu as pltpu
from jax.experimental.pallas import tpu_sc as plsc

NC, NS, LANES = 2, 16, 16
NW = NC * NS


def _mesh():
    return plsc.VectorSubcoreMesh(core_axis_name="c", subcore_axis_name="s")


def _iota16():
    return lax.iota(jnp.int32, 16)


def _make_ephase(E_pad, N_pad, heads16):
    C = 128
    SLAB = E_pad // NW
    n_micro = SLAB // C
    STRIPE = N_pad // NS
    if heads16:
        ex_shape, s_shape = (E_pad, 16), (NC, N_pad, 16)
        exb_t = pltpu.VMEM((C, 16), jnp.float32)
        ab_t = pltpu.VMEM((C, 16), jnp.float32)
        zb_t = pltpu.VMEM((128, 16), jnp.float32)
    else:
        ex_shape, s_shape = (E_pad,), (NC, N_pad)
        exb_t = pltpu.VMEM((C,), jnp.float32)
        ab_t = pltpu.VMEM((C,), jnp.float32)
        zb_t = pltpu.VMEM((128,), jnp.float32)

    @functools.partial(
        pl.kernel,
        out_type=(jax.ShapeDtypeStruct(ex_shape, jnp.float32),
                  jax.ShapeDtypeStruct(s_shape, jnp.float32)),
        mesh=_mesh(),
        compiler_params=pltpu.CompilerParams(use_tc_tiling_on_sc=False),
        scratch_types=[
            pltpu.VMEM((C,), jnp.int32),
            pltpu.VMEM((C,), jnp.int32),
            ab_t, ab_t, exb_t, zb_t,
            (pltpu.VMEM_SHARED((N_pad, 16), jnp.float32) if heads16
             else pltpu.VMEM_SHARED((N_pad,), jnp.float32)),
            pltpu.SemaphoreType.DMA,
        ],
    )
    def k(src_h, dst_h, as_h, ad_h, ex_h, sp_h,
          srcb, dstb, asb, adb, exb, zbuf, sacc, sem):
        cid = lax.axis_index("c")
        sid = lax.axis_index("s")
        wid = sid * NC + cid
        base = wid * SLAB
        if heads16:
            for r in range(128):
                zbuf[r, :] = jnp.zeros((16,), jnp.float32)
        else:
            for r in range(8):
                zbuf[pl.ds(r * 16, 16)] = jnp.zeros((16,), jnp.float32)

        def zrow(b, _):
            if heads16:
                pltpu.sync_copy(zbuf, sacc.at[pl.ds(sid * STRIPE + b * 128, 128)])
            else:
                pltpu.sync_copy(zbuf, sacc.at[pl.ds(sid * STRIPE + b * 128, 128)])
            return 0

        lax.fori_loop(0, STRIPE // 128, zrow, 0)
        plsc.subcore_barrier()

        def micro(i, _):
            off = base + i * C
            pltpu.sync_copy(src_h.at[pl.ds(off, C)], srcb)
            pltpu.sync_copy(dst_h.at[pl.ds(off, C)], dstb)
            pltpu.async_copy(as_h.at[srcb], asb, sem).wait()
            pltpu.async_copy(ad_h.at[dstb], adb, sem).wait()
            if heads16:
                def edge(e, _):
                    z = asb[e, :] + adb[e, :]
                    z = jnp.maximum(z, 0.2 * z)
                    exb[e, :] = jnp.exp(z)
                    return 0
                lax.fori_loop(0, C, edge, 0)
            else:
                for q in range(C // 16):
                    z = asb[pl.ds(q * 16, 16)] + adb[pl.ds(q * 16, 16)]
                    z = jnp.maximum(z, 0.2 * z)
                    exb[pl.ds(q * 16, 16)] = jnp.exp(z)
            pltpu.sync_copy(exb, ex_h.at[pl.ds(off, C)])
            pltpu.sync_copy(exb, sacc.at[dstb], add=True)
            return 0

        lax.fori_loop(0, n_micro, micro, 0)
        plsc.subcore_barrier()
        if heads16:
            pltpu.sync_copy(sacc.at[pl.ds(sid * STRIPE, STRIPE)],
                            sp_h.at[cid, pl.ds(sid * STRIPE, STRIPE)])
        else:
            pltpu.sync_copy(sacc.at[pl.ds(sid * STRIPE, STRIPE)],
                            sp_h.at[cid, pl.ds(sid * STRIPE, STRIPE)])

    return k


def _make_agg(E_pad, N_pad, F, heads, hd_stride, RPS):
    MC = 64
    SCAN = 2048
    SLAB = E_pad // NS
    NPASS = N_pad // (NC * RPS)
    STRIPE = RPS // NS
    heads16 = heads > 1
    exb_shape = (MC, 16) if heads16 else (MC,)

    @functools.partial(
        pl.kernel,
        out_type=jax.ShapeDtypeStruct((N_pad, F), jnp.float32),
        mesh=_mesh(),
        compiler_params=pltpu.CompilerParams(
            use_tc_tiling_on_sc=False, needs_layout_passes=False),
        scratch_types=[
            pltpu.VMEM((SCAN,), jnp.int32),
            pltpu.VMEM((SCAN + 96,), jnp.int32),
            pltpu.VMEM((MC,), jnp.int32),
            pltpu.VMEM((MC,), jnp.int32),
            pltpu.VMEM((MC,), jnp.int32),
            pltpu.VMEM(exb_shape, jnp.float32),
            pltpu.VMEM((MC, F), jnp.float32),
            pltpu.VMEM((4, F), jnp.float32),
            pltpu.VMEM_SHARED((RPS + 8, F), jnp.float32),
            pltpu.SemaphoreType.DMA,
        ],
    )
    def k(src_h, dst_h, ex_h, h_h, out_h,
          dstc, lst, eidx, srcv, sidx, exb, hb, zbuf, acc, sem):
        cid = lax.axis_index("c")
        sid = lax.axis_index("s")
        slab0 = sid * SLAB
        iota = _iota16()
        for r in range(4):
            for j in range(F // 16):
                zbuf[r, pl.ds(j * 16, 16)] = jnp.zeros((16,), jnp.float32)

        def zero_stripe():
            def zrow(b, _):
                pltpu.sync_copy(zbuf, acc.at[pl.ds(sid * STRIPE + b * 4, 4)])
                return 0
            lax.fori_loop(0, STRIPE // 4, zrow, 0)

        zero_stripe()

        @pl.when(sid == 0)
        def _():
            pltpu.sync_copy(zbuf, acc.at[pl.ds(RPS, 4)])
            pltpu.sync_copy(zbuf, acc.at[pl.ds(RPS + 4, 4)])

        plsc.subcore_barrier()

        def do_pass(p, _):
            lo = p * (NC * RPS) + cid * RPS
            hi = lo + RPS

            def micro(q, _2):
                qb = q * MC
                for kk in range(MC // 16):
                    w = lst[pl.ds(qb + kk * 16, 16)]
                    eidx[pl.ds(kk * 16, 16)] = (w & 65535) + slab0
                    sidx[pl.ds(kk * 16, 16)] = lax.shift_right_logical(w, 16)
                pltpu.async_copy(src_h.at[eidx], srcv, sem).wait()
                pltpu.async_copy(ex_h.at[eidx], exb, sem).wait()
                pltpu.async_copy(h_h.at[srcv], hb, sem).wait()

                def edge(i, _3):
                    if heads16:
                        for hd in range(heads):
                            wv = plsc.load_gather(
                                exb, [jnp.full((16,), i, jnp.int32),
                                      jnp.full((16,), hd, jnp.int32)])
                            for j in range(hd_stride // 16):
                                col = hd * hd_stride + j * 16
                                hb[i, pl.ds(col, 16)] = hb[i, pl.ds(col, 16)] * wv
                    else:
                        wv = plsc.load_gather(exb, [jnp.full((16,), i, jnp.int32)])
                        for j in range(F // 16):
                            hb[i, pl.ds(j * 16, 16)] = hb[i, pl.ds(j * 16, 16)] * wv
                    return 0

                lax.fori_loop(0, MC, edge, 0)
                pltpu.sync_copy(hb, acc.at[sidx], add=True)
                return 0

            def scan_chunk(t, _2):
                pltpu.sync_copy(dst_h.at[pl.ds(slab0 + t * SCAN, SCAN)], dstc)

                def vec(kk, cur2):
                    d = dstc[pl.ds(kk * 16, 16)]
                    m = (d >= lo) & (d < hi)
                    leid = iota + (t * SCAN + kk * 16)
                    w = leid | lax.shift_left(d - lo, 16)
                    c = lax.cumsum(m.astype(jnp.int32))
                    pos = jnp.where(m, cur2 + c - 1,
                                    jnp.full((16,), SCAN + 80, jnp.int32))
                    plsc.store_scatter(lst, [pos], w)
                    return cur2 + jnp.max(c)

                cnt = lax.fori_loop(0, SCAN // 16, vec, 0)
                padw = (iota + t * SCAN) | lax.shift_left(
                    RPS + (iota & 7), 16)
                for j in range(5):
                    lst[pl.ds(cnt + j * 16, 16)] = padw
                nq = (cnt + MC - 1) // MC
                lax.fori_loop(0, nq, micro, 0)
                return 0

            lax.fori_loop(0, SLAB // SCAN, scan_chunk, 0)
            plsc.subcore_barrier()
            row0 = p * (NC * RPS) + cid * RPS + sid * STRIPE
            pltpu.sync_copy(acc.at[pl.ds(sid * STRIPE, STRIPE)],
                            out_h.at[pl.ds(row0, STRIPE)])
            zero_stripe()
            plsc.subcore_barrier()
            return 0

        lax.fori_loop(0, NPASS, do_pass, 0)

    return k


def _mm1_body(N, xb_ref, w_ref, as_ref, ad_ref, h_ref, at_ref, dt_ref):
    blk = xb_ref.shape[0]
    h = jnp.dot(xb_ref[...], w_ref[...], preferred_element_type=jnp.float32)
    h3 = h.reshape(blk, 10, 78)
    a_s = jnp.sum(h3 * as_ref[...][None], axis=-1)
    a_d = jnp.sum(h3 * ad_ref[...][None], axis=-1)
    a_s = jnp.concatenate([a_s, jnp.zeros((blk, 6), jnp.float32)], axis=1)
    a_d = jnp.concatenate([a_d, jnp.full((blk, 6), -1e30, jnp.float32)], axis=1)
    rows = pl.program_id(0) * blk + lax.broadcasted_iota(jnp.int32, (blk, 1), 0)
    a_d = jnp.where(rows < N, a_d, -1e30)
    h_ref[...] = jnp.pad(h3, ((0, 0), (0, 0), (0, 2))).reshape(blk, 800)
    at_ref[...] = a_s
    dt_ref[...] = a_d


def _mm1(x_pad, W1, as1, ad1, N):
    N_pad = x_pad.shape[0]
    blk = 512
    grid = N_pad // blk
    return pl.pallas_call(
        functools.partial(_mm1_body, N),
        grid=(grid,),
        in_specs=[
            pl.BlockSpec((blk, 78), lambda i: (i, 0)),
            pl.BlockSpec((78, 780), lambda i: (0, 0)),
            pl.BlockSpec((10, 78), lambda i: (0, 0)),
            pl.BlockSpec((10, 78), lambda i: (0, 0)),
        ],
        out_specs=[
            pl.BlockSpec((blk, 800), lambda i: (i, 0)),
            pl.BlockSpec((blk, 16), lambda i: (i, 0)),
            pl.BlockSpec((blk, 16), lambda i: (i, 0)),
        ],
        out_shape=[
            jax.ShapeDtypeStruct((N_pad, 800), jnp.float32),
            jax.ShapeDtypeStruct((N_pad, 16), jnp.float32),
            jax.ShapeDtypeStruct((N_pad, 16), jnp.float32),
        ],
    )(x_pad, W1, as1, ad1)


def _mm2_body(N, xb_ref, w_ref, as_ref, ad_ref, h_ref, at_ref, dt_ref):
    blk = xb_ref.shape[0]
    h = jnp.dot(xb_ref[...], w_ref[...], preferred_element_type=jnp.float32)
    a_s = jnp.sum(h * as_ref[...], axis=-1, keepdims=True)
    a_d = jnp.sum(h * ad_ref[...], axis=-1, keepdims=True)
    rows = pl.program_id(0) * blk + lax.broadcasted_iota(jnp.int32, (blk, 1), 0)
    a_d = jnp.where(rows < N, a_d, -1e30)
    h_ref[...] = h
    at_ref[...] = a_s
    dt_ref[...] = a_d


def _mm2(h1_pad, W2, as2, ad2, N):
    N_pad = h1_pad.shape[0]
    blk = 512
    grid = N_pad // blk
    return pl.pallas_call(
        functools.partial(_mm2_body, N),
        grid=(grid,),
        in_specs=[
            pl.BlockSpec((blk, 780), lambda i: (i, 0)),
            pl.BlockSpec((780, 128), lambda i: (0, 0)),
            pl.BlockSpec((1, 128), lambda i: (0, 0)),
            pl.BlockSpec((1, 128), lambda i: (0, 0)),
        ],
        out_specs=[
            pl.BlockSpec((blk, 128), lambda i: (i, 0)),
            pl.BlockSpec((blk, 1), lambda i: (i, 0)),
            pl.BlockSpec((blk, 1), lambda i: (i, 0)),
        ],
        out_shape=[
            jax.ShapeDtypeStruct((N_pad, 128), jnp.float32),
            jax.ShapeDtypeStruct((N_pad, 1), jnp.float32),
            jax.ShapeDtypeStruct((N_pad, 1), jnp.float32),
        ],
    )(h1_pad, W2, as2, ad2)


def _head_body(xc_ref, w1_ref, b1_ref, w2_ref, b2_ref, wo_ref, bo_ref, out_ref):
    h1 = jnp.maximum(
        jnp.dot(xc_ref[...], w1_ref[...], preferred_element_type=jnp.float32)
        + b1_ref[...], 0.0)
    h2 = jnp.maximum(
        jnp.dot(h1, w2_ref[...], preferred_element_type=jnp.float32)
        + b2_ref[...], 0.0)
    out_ref[...] = (
        jnp.dot(h2, wo_ref[...], preferred_element_type=jnp.float32) + bo_ref[...])


def _head(xc, W_fc1, b_fc1, W_fc2, b_fc2, W_out, b_out):
    B = xc.shape[0]
    return pl.pallas_call(
        _head_body,
        out_shape=jax.ShapeDtypeStruct((B, 1), jnp.float32),
    )(xc, W_fc1, b_fc1.reshape(1, -1), W_fc2, b_fc2.reshape(1, -1),
      W_out, b_out.reshape(1, -1))


def kernel(x, edge_index, batch, target, W1, as1, ad1, b1, W2, as2, ad2, b2,
           Wg, bg, emb, cw, cb, Wxt, bxt, W_fc1, b_fc1, W_fc2, b_fc2,
           W_out, b_out):
    N = x.shape[0]
    E = edge_index.shape[1]
    B = target.shape[0]
    N_pad = 53248
    E_tot = E + N
    SLAB32 = -(-E_tot // (NW * 2048)) * 2048
    E_pad = NW * SLAB32

    loop = jnp.arange(N, dtype=edge_index.dtype)
    src = jnp.concatenate([edge_index[0], loop])
    dst = jnp.concatenate([edge_index[1], loop])
    npe = E_pad - E_tot
    src_p = jnp.concatenate([src, (jnp.arange(npe, dtype=jnp.int32) * 37) % N])
    dst_p = jnp.concatenate([dst, jnp.full((npe,), N, jnp.int32)])

    x_pad = jnp.pad(x, ((0, N_pad - N), (0, 0)))
    h1p, as1t, ad1t = _mm1(x_pad, W1, as1, ad1, N)

    eph1 = _make_ephase(E_pad, N_pad, True)
    ex1, s1p = eph1(src_p, dst_p, as1t, ad1t)
    agg1 = _make_agg(E_pad, N_pad, 800, 10, 80, 1024)
    outr1 = agg1(src_p, dst_p, ex1, h1p)

    s1 = (s1p[0] + s1p[1])[:N, :10]
    h1 = outr1.reshape(N_pad, 10, 80)[:N, :, :78] / (s1[:, :, None] + 1e-16)
    h1 = jax.nn.elu(h1.reshape(N, 780) + b1)

    h1a = jnp.pad(h1, ((0, N_pad - N), (0, 0)))
    h2p, as2t, ad2t = _mm2(h1a, W2, as2, ad2, N)
    eph2 = _make_ephase(E_pad, N_pad, False)
    ex2, s2p = eph2(src_p, dst_p, as2t.reshape(-1), ad2t.reshape(-1))
    agg2 = _make_agg(E_pad, N_pad, 128, 1, 128, 13312)
    outr2 = agg2(src_p, dst_p, ex2, h2p)

    s2 = (s2p[0] + s2p[1])[:N]
    h2 = outr2[:N] / (s2[:, None] + 1e-16) + b2
    h = jax.nn.relu(h2)

    g = jax.ops.segment_max(h, batch, num_segments=B)
    g = jax.nn.relu(g @ Wg + bg)
    emb_t = jnp.take(emb, target, axis=0)
    conv = jax.lax.conv_general_dilated(
        emb_t, cw, (1,), 'VALID', dimension_numbers=('NCH', 'OIH', 'NCH'))
    conv = jax.nn.relu(conv + cb[None, :, None])
    xt = conv.reshape(B, 32 * 121) @ Wxt + bxt
    xc = jnp.concatenate([g, xt], axis=1)
    return _head(xc, W_fc1, b_fc1, W_fc2, b_fc2, W_out, b_out)


# speedup vs baseline: 1.1598x; 1.1598x over previous
import functools

import jax
import jax.numpy as jnp
from jax import lax
from jax.experimental import pallas as pl
from jax.experimental.pallas import tpu as pltpu
from jax.experimental.pallas import tpu_sc as plsc

NC, NS, LANES = 2, 16, 16
NW = NC * NS


def _mesh():
    return plsc.VectorSubcoreMesh(core_axis_name="c", subcore_axis_name="s")


def _iota16():
    return lax.iota(jnp.int32, 16)


def _make_ephase(E_pad, N_pad, heads16):
    C = 128
    SLAB = E_pad // NW
    n_micro = SLAB // C
    STRIPE = N_pad // NS
    if heads16:
        ex_shape, s_shape = (E_pad, 16), (NC, N_pad, 16)
        exb_t = pltpu.VMEM((C, 16), jnp.float32)
        ab_t = pltpu.VMEM((C, 16), jnp.float32)
        zb_t = pltpu.VMEM((128, 16), jnp.float32)
    else:
        ex_shape, s_shape = (E_pad,), (NC, N_pad)
        exb_t = pltpu.VMEM((C,), jnp.float32)
        ab_t = pltpu.VMEM((C,), jnp.float32)
        zb_t = pltpu.VMEM((128,), jnp.float32)

    @functools.partial(
        pl.kernel,
        out_type=(jax.ShapeDtypeStruct(ex_shape, jnp.float32),
                  jax.ShapeDtypeStruct(s_shape, jnp.float32)),
        mesh=_mesh(),
        compiler_params=pltpu.CompilerParams(use_tc_tiling_on_sc=False),
        scratch_types=[
            pltpu.VMEM((C,), jnp.int32),
            pltpu.VMEM((C,), jnp.int32),
            ab_t, ab_t, exb_t, zb_t,
            (pltpu.VMEM_SHARED((N_pad, 16), jnp.float32) if heads16
             else pltpu.VMEM_SHARED((N_pad,), jnp.float32)),
            pltpu.SemaphoreType.DMA,
        ],
    )
    def k(src_h, dst_h, as_h, ad_h, ex_h, sp_h,
          srcb, dstb, asb, adb, exb, zbuf, sacc, sem):
        cid = lax.axis_index("c")
        sid = lax.axis_index("s")
        wid = sid * NC + cid
        base = wid * SLAB
        if heads16:
            for r in range(128):
                zbuf[r, :] = jnp.zeros((16,), jnp.float32)
        else:
            for r in range(8):
                zbuf[pl.ds(r * 16, 16)] = jnp.zeros((16,), jnp.float32)

        def zrow(b, _):
            if heads16:
                pltpu.sync_copy(zbuf, sacc.at[pl.ds(sid * STRIPE + b * 128, 128)])
            else:
                pltpu.sync_copy(zbuf, sacc.at[pl.ds(sid * STRIPE + b * 128, 128)])
            return 0

        lax.fori_loop(0, STRIPE // 128, zrow, 0)
        plsc.subcore_barrier()

        def micro(i, _):
            off = base + i * C
            pltpu.sync_copy(src_h.at[pl.ds(off, C)], srcb)
            pltpu.sync_copy(dst_h.at[pl.ds(off, C)], dstb)
            pltpu.async_copy(as_h.at[srcb], asb, sem).wait()
            pltpu.async_copy(ad_h.at[dstb], adb, sem).wait()
            if heads16:
                def edge(e, _):
                    z = asb[e, :] + adb[e, :]
                    z = jnp.maximum(z, 0.2 * z)
                    exb[e, :] = jnp.exp(z)
                    return 0
                lax.fori_loop(0, C, edge, 0)
            else:
                for q in range(C // 16):
                    z = asb[pl.ds(q * 16, 16)] + adb[pl.ds(q * 16, 16)]
                    z = jnp.maximum(z, 0.2 * z)
                    exb[pl.ds(q * 16, 16)] = jnp.exp(z)
            pltpu.sync_copy(exb, ex_h.at[pl.ds(off, C)])
            pltpu.sync_copy(exb, sacc.at[dstb], add=True)
            return 0

        lax.fori_loop(0, n_micro, micro, 0)
        plsc.subcore_barrier()
        if heads16:
            pltpu.sync_copy(sacc.at[pl.ds(sid * STRIPE, STRIPE)],
                            sp_h.at[cid, pl.ds(sid * STRIPE, STRIPE)])
        else:
            pltpu.sync_copy(sacc.at[pl.ds(sid * STRIPE, STRIPE)],
                            sp_h.at[cid, pl.ds(sid * STRIPE, STRIPE)])

    return k


def _make_agg(E_pad, N_pad, F, heads, hd_stride, RPS, SCAN):
    MC = 64
    SLAB = E_pad // NS
    NPASS = N_pad // (NC * RPS)
    STRIPE = RPS // NS
    heads16 = heads > 1
    exb_shape = (MC, 16) if heads16 else (MC,)

    @functools.partial(
        pl.kernel,
        out_type=jax.ShapeDtypeStruct((N_pad, F), jnp.float32),
        mesh=_mesh(),
        compiler_params=pltpu.CompilerParams(
            use_tc_tiling_on_sc=False, needs_layout_passes=False),
        scratch_types=[
            pltpu.VMEM((SCAN,), jnp.int32),
            pltpu.VMEM((SCAN,), jnp.int32),
            pltpu.VMEM((SCAN + 96,), jnp.int32),
            pltpu.VMEM((MC,), jnp.int32),
            pltpu.VMEM((MC,), jnp.int32),
            pltpu.VMEM((MC,), jnp.int32),
            pltpu.VMEM(exb_shape, jnp.float32),
            pltpu.VMEM((MC, F), jnp.float32),
            pltpu.VMEM((4, F), jnp.float32),
            pltpu.VMEM_SHARED((RPS + 8, F), jnp.float32),
            pltpu.SemaphoreType.DMA,
            pltpu.SemaphoreType.DMA,
        ],
    )
    def k(src_h, dst_h, ex_h, h_h, out_h,
          dstc, srcc, lst, eidx, srcv, sidx, exb, hb, zbuf, acc, sem, sem2):
        cid = lax.axis_index("c")
        sid = lax.axis_index("s")
        slab0 = sid * SLAB
        iota = _iota16()
        for r in range(4):
            for j in range(F // 16):
                zbuf[r, pl.ds(j * 16, 16)] = jnp.zeros((16,), jnp.float32)

        def zero_stripe():
            def zrow(b, _):
                pltpu.sync_copy(zbuf, acc.at[pl.ds(sid * STRIPE + b * 4, 4)])
                return 0
            lax.fori_loop(0, STRIPE // 4, zrow, 0)

        zero_stripe()

        @pl.when(sid == 0)
        def _():
            pltpu.sync_copy(zbuf, acc.at[pl.ds(RPS, 4)])
            pltpu.sync_copy(zbuf, acc.at[pl.ds(RPS + 4, 4)])

        plsc.subcore_barrier()

        def do_pass(p, _):
            lo = p * (NC * RPS) + cid * RPS
            hi = lo + RPS

            def micro(q, targs):
                t = targs
                qb = q * MC
                for kk in range(MC // 16):
                    w = lst[pl.ds(qb + kk * 16, 16)]
                    lid = w & 8191
                    eidx[pl.ds(kk * 16, 16)] = lid + (slab0 + t * SCAN)
                    sidx[pl.ds(kk * 16, 16)] = lax.shift_right_logical(w, 13)
                    srcv[pl.ds(kk * 16, 16)] = plsc.load_gather(srcc, [lid])
                c1 = pltpu.async_copy(ex_h.at[eidx], exb, sem)
                c2 = pltpu.async_copy(h_h.at[srcv], hb, sem2)
                c1.wait()
                c2.wait()

                def edge(i, _3):
                    if heads16:
                        for hd in range(heads):
                            wv = plsc.load_gather(
                                exb, [jnp.full((16,), i, jnp.int32),
                                      jnp.full((16,), hd, jnp.int32)])
                            for j in range(hd_stride // 16):
                                col = hd * hd_stride + j * 16
                                hb[i, pl.ds(col, 16)] = hb[i, pl.ds(col, 16)] * wv
                    else:
                        wv = plsc.load_gather(exb, [jnp.full((16,), i, jnp.int32)])
                        for j in range(F // 16):
                            hb[i, pl.ds(j * 16, 16)] = hb[i, pl.ds(j * 16, 16)] * wv
                    return 0

                lax.fori_loop(0, MC, edge, 0)
                pltpu.sync_copy(hb, acc.at[sidx], add=True)
                return targs

            def scan_chunk(t, _2):
                pltpu.sync_copy(dst_h.at[pl.ds(slab0 + t * SCAN, SCAN)], dstc)
                pltpu.sync_copy(src_h.at[pl.ds(slab0 + t * SCAN, SCAN)], srcc)

                def vec(kk, cur2):
                    d = dstc[pl.ds(kk * 16, 16)]
                    m = (d >= lo) & (d < hi)
                    lid = iota + kk * 16
                    w = lid | lax.shift_left(d - lo, 13)
                    c = lax.cumsum(m.astype(jnp.int32))
                    pos = jnp.where(m, cur2 + c - 1,
                                    jnp.full((16,), SCAN + 80, jnp.int32))
                    plsc.store_scatter(lst, [pos], w)
                    return cur2 + jnp.max(c)

                cnt = lax.fori_loop(0, SCAN // 16, vec, 0)
                padw = iota | lax.shift_left(RPS + (iota & 7), 13)
                for j in range(5):
                    lst[pl.ds(cnt + j * 16, 16)] = padw
                nq = (cnt + MC - 1) // MC
                lax.fori_loop(0, nq, micro, t)
                return 0

            lax.fori_loop(0, SLAB // SCAN, scan_chunk, 0)
            plsc.subcore_barrier()
            row0 = p * (NC * RPS) + cid * RPS + sid * STRIPE
            pltpu.sync_copy(acc.at[pl.ds(sid * STRIPE, STRIPE)],
                            out_h.at[pl.ds(row0, STRIPE)])
            zero_stripe()
            plsc.subcore_barrier()
            return 0

        lax.fori_loop(0, NPASS, do_pass, 0)

    return k


def _mm1_body(N, xb_ref, w_ref, as_ref, ad_ref, h_ref, at_ref, dt_ref):
    blk = xb_ref.shape[0]
    h = jnp.dot(xb_ref[...], w_ref[...], preferred_element_type=jnp.float32)
    h3 = h.reshape(blk, 10, 78)
    a_s = jnp.sum(h3 * as_ref[...][None], axis=-1)
    a_d = jnp.sum(h3 * ad_ref[...][None], axis=-1)
    a_s = jnp.concatenate([a_s, jnp.zeros((blk, 6), jnp.float32)], axis=1)
    a_d = jnp.concatenate([a_d, jnp.full((blk, 6), -1e30, jnp.float32)], axis=1)
    rows = pl.program_id(0) * blk + lax.broadcasted_iota(jnp.int32, (blk, 1), 0)
    a_d = jnp.where(rows < N, a_d, -1e30)
    h_ref[...] = jnp.pad(h3, ((0, 0), (0, 0), (0, 2))).reshape(blk, 800)
    at_ref[...] = a_s
    dt_ref[...] = a_d


def _mm1(x_pad, W1, as1, ad1, N):
    N_pad = x_pad.shape[0]
    blk = 512
    grid = N_pad // blk
    return pl.pallas_call(
        functools.partial(_mm1_body, N),
        grid=(grid,),
        in_specs=[
            pl.BlockSpec((blk, 78), lambda i: (i, 0)),
            pl.BlockSpec((78, 780), lambda i: (0, 0)),
            pl.BlockSpec((10, 78), lambda i: (0, 0)),
            pl.BlockSpec((10, 78), lambda i: (0, 0)),
        ],
        out_specs=[
            pl.BlockSpec((blk, 800), lambda i: (i, 0)),
            pl.BlockSpec((blk, 16), lambda i: (i, 0)),
            pl.BlockSpec((blk, 16), lambda i: (i, 0)),
        ],
        out_shape=[
            jax.ShapeDtypeStruct((N_pad, 800), jnp.float32),
            jax.ShapeDtypeStruct((N_pad, 16), jnp.float32),
            jax.ShapeDtypeStruct((N_pad, 16), jnp.float32),
        ],
    )(x_pad, W1, as1, ad1)


def _mm2_body(N, xb_ref, w_ref, as_ref, ad_ref, h_ref, at_ref, dt_ref):
    blk = xb_ref.shape[0]
    h = jnp.dot(xb_ref[...], w_ref[...], preferred_element_type=jnp.float32)
    a_s = jnp.sum(h * as_ref[...], axis=-1, keepdims=True)
    a_d = jnp.sum(h * ad_ref[...], axis=-1, keepdims=True)
    rows = pl.program_id(0) * blk + lax.broadcasted_iota(jnp.int32, (blk, 1), 0)
    a_d = jnp.where(rows < N, a_d, -1e30)
    h_ref[...] = h
    at_ref[...] = a_s
    dt_ref[...] = a_d


def _mm2(h1_pad, W2, as2, ad2, N):
    N_pad = h1_pad.shape[0]
    blk = 512
    grid = N_pad // blk
    return pl.pallas_call(
        functools.partial(_mm2_body, N),
        grid=(grid,),
        in_specs=[
            pl.BlockSpec((blk, 780), lambda i: (i, 0)),
            pl.BlockSpec((780, 128), lambda i: (0, 0)),
            pl.BlockSpec((1, 128), lambda i: (0, 0)),
            pl.BlockSpec((1, 128), lambda i: (0, 0)),
        ],
        out_specs=[
            pl.BlockSpec((blk, 128), lambda i: (i, 0)),
            pl.BlockSpec((blk, 1), lambda i: (i, 0)),
            pl.BlockSpec((blk, 1), lambda i: (i, 0)),
        ],
        out_shape=[
            jax.ShapeDtypeStruct((N_pad, 128), jnp.float32),
            jax.ShapeDtypeStruct((N_pad, 1), jnp.float32),
            jax.ShapeDtypeStruct((N_pad, 1), jnp.float32),
        ],
    )(h1_pad, W2, as2, ad2)


def _head_body(xc_ref, w1_ref, b1_ref, w2_ref, b2_ref, wo_ref, bo_ref, out_ref):
    h1 = jnp.maximum(
        jnp.dot(xc_ref[...], w1_ref[...], preferred_element_type=jnp.float32)
        + b1_ref[...], 0.0)
    h2 = jnp.maximum(
        jnp.dot(h1, w2_ref[...], preferred_element_type=jnp.float32)
        + b2_ref[...], 0.0)
    out_ref[...] = (
        jnp.dot(h2, wo_ref[...], preferred_element_type=jnp.float32) + bo_ref[...])


def _head(xc, W_fc1, b_fc1, W_fc2, b_fc2, W_out, b_out):
    B = xc.shape[0]
    return pl.pallas_call(
        _head_body,
        out_shape=jax.ShapeDtypeStruct((B, 1), jnp.float32),
    )(xc, W_fc1, b_fc1.reshape(1, -1), W_fc2, b_fc2.reshape(1, -1),
      W_out, b_out.reshape(1, -1))


def kernel(x, edge_index, batch, target, W1, as1, ad1, b1, W2, as2, ad2, b2,
           Wg, bg, emb, cw, cb, Wxt, bxt, W_fc1, b_fc1, W_fc2, b_fc2,
           W_out, b_out):
    N = x.shape[0]
    E = edge_index.shape[1]
    B = target.shape[0]
    N_pad = 53248
    E_tot = E + N
    SLAB32 = -(-E_tot // (NW * 2048)) * 2048
    E_pad = NW * SLAB32

    loop = jnp.arange(N, dtype=edge_index.dtype)
    src = jnp.concatenate([edge_index[0], loop])
    dst = jnp.concatenate([edge_index[1], loop])
    npe = E_pad - E_tot
    src_p = jnp.concatenate([src, (jnp.arange(npe, dtype=jnp.int32) * 37) % N])
    dst_p = jnp.concatenate([dst, jnp.full((npe,), N, jnp.int32)])

    x_pad = jnp.pad(x, ((0, N_pad - N), (0, 0)))
    h1p, as1t, ad1t = _mm1(x_pad, W1, as1, ad1, N)

    eph1 = _make_ephase(E_pad, N_pad, True)
    ex1, s1p = eph1(src_p, dst_p, as1t, ad1t)
    agg1 = _make_agg(E_pad, N_pad, 800, 10, 80, 1024, 6656)
    outr1 = agg1(src_p, dst_p, ex1, h1p)

    s1 = (s1p[0] + s1p[1])[:N, :10]
    h1 = outr1.reshape(N_pad, 10, 80)[:N, :, :78] / (s1[:, :, None] + 1e-16)
    h1 = jax.nn.elu(h1.reshape(N, 780) + b1)

    h1a = jnp.pad(h1, ((0, N_pad - N), (0, 0)))
    h2p, as2t, ad2t = _mm2(h1a, W2, as2, ad2, N)
    eph2 = _make_ephase(E_pad, N_pad, False)
    ex2, s2p = eph2(src_p, dst_p, as2t.reshape(-1), ad2t.reshape(-1))
    agg2 = _make_agg(E_pad, N_pad, 128, 1, 128, 13312, 4096)
    outr2 = agg2(src_p, dst_p, ex2, h2p)

    s2 = (s2p[0] + s2p[1])[:N]
    h2 = outr2[:N] / (s2[:, None] + 1e-16) + b2
    h = jax.nn.relu(h2)

    g = jax.ops.segment_max(h, batch, num_segments=B)
    g = jax.nn.relu(g @ Wg + bg)
    emb_t = jnp.take(emb, target, axis=0)
    conv = jax.lax.conv_general_dilated(
        emb_t, cw, (1,), 'VALID', dimension_numbers=('NCH', 'OIH', 'NCH'))
    conv = jax.nn.relu(conv + cb[None, :, None])
    xt = conv.reshape(B, 32 * 121) @ Wxt + bxt
    xc = jnp.concatenate([g, xt], axis=1)
    return _head(xc, W_fc1, b_fc1, W_fc2, b_fc2, W_out, b_out)


# speedup vs baseline: 1.2914x; 1.1134x over previous
import functools

import jax
import jax.numpy as jnp
from jax import lax
from jax.experimental import pallas as pl
from jax.experimental.pallas import tpu as pltpu
from jax.experimental.pallas import tpu_sc as plsc

NC, NS, LANES = 2, 16, 16
NW = NC * NS


def _mesh():
    return plsc.VectorSubcoreMesh(core_axis_name="c", subcore_axis_name="s")


def _iota16():
    return lax.iota(jnp.int32, 16)


def _make_ephase(E_pad, N_pad, heads16):
    C = 128
    SLAB = E_pad // NW
    n_micro = SLAB // C
    STRIPE = N_pad // NS
    if heads16:
        ex_shape, s_shape = (E_pad, 16), (NC, N_pad, 16)
        exb_t = pltpu.VMEM((C, 16), jnp.float32)
        ab_t = pltpu.VMEM((C, 16), jnp.float32)
        zb_t = pltpu.VMEM((128, 16), jnp.float32)
    else:
        ex_shape, s_shape = (E_pad,), (NC, N_pad)
        exb_t = pltpu.VMEM((C,), jnp.float32)
        ab_t = pltpu.VMEM((C,), jnp.float32)
        zb_t = pltpu.VMEM((128,), jnp.float32)

    @functools.partial(
        pl.kernel,
        out_type=(jax.ShapeDtypeStruct(ex_shape, jnp.float32),
                  jax.ShapeDtypeStruct(s_shape, jnp.float32)),
        mesh=_mesh(),
        compiler_params=pltpu.CompilerParams(use_tc_tiling_on_sc=False),
        scratch_types=[
            pltpu.VMEM((C,), jnp.int32),
            pltpu.VMEM((C,), jnp.int32),
            ab_t, ab_t, exb_t, zb_t,
            (pltpu.VMEM_SHARED((N_pad, 16), jnp.float32) if heads16
             else pltpu.VMEM_SHARED((N_pad,), jnp.float32)),
            pltpu.SemaphoreType.DMA,
        ],
    )
    def k(src_h, dst_h, as_h, ad_h, ex_h, sp_h,
          srcb, dstb, asb, adb, exb, zbuf, sacc, sem):
        cid = lax.axis_index("c")
        sid = lax.axis_index("s")
        wid = sid * NC + cid
        base = wid * SLAB
        if heads16:
            for r in range(128):
                zbuf[r, :] = jnp.zeros((16,), jnp.float32)
        else:
            for r in range(8):
                zbuf[pl.ds(r * 16, 16)] = jnp.zeros((16,), jnp.float32)

        def zrow(b, _):
            if heads16:
                pltpu.sync_copy(zbuf, sacc.at[pl.ds(sid * STRIPE + b * 128, 128)])
            else:
                pltpu.sync_copy(zbuf, sacc.at[pl.ds(sid * STRIPE + b * 128, 128)])
            return 0

        lax.fori_loop(0, STRIPE // 128, zrow, 0)
        plsc.subcore_barrier()

        def micro(i, _):
            off = base + i * C
            pltpu.sync_copy(src_h.at[pl.ds(off, C)], srcb)
            pltpu.sync_copy(dst_h.at[pl.ds(off, C)], dstb)
            pltpu.async_copy(as_h.at[srcb], asb, sem).wait()
            pltpu.async_copy(ad_h.at[dstb], adb, sem).wait()
            if heads16:
                def edge(e, _):
                    z = asb[e, :] + adb[e, :]
                    z = jnp.maximum(z, 0.2 * z)
                    exb[e, :] = jnp.exp(z)
                    return 0
                lax.fori_loop(0, C, edge, 0)
            else:
                for q in range(C // 16):
                    z = asb[pl.ds(q * 16, 16)] + adb[pl.ds(q * 16, 16)]
                    z = jnp.maximum(z, 0.2 * z)
                    exb[pl.ds(q * 16, 16)] = jnp.exp(z)
            pltpu.sync_copy(exb, ex_h.at[pl.ds(off, C)])
            pltpu.sync_copy(exb, sacc.at[dstb], add=True)
            return 0

        lax.fori_loop(0, n_micro, micro, 0)
        plsc.subcore_barrier()
        if heads16:
            pltpu.sync_copy(sacc.at[pl.ds(sid * STRIPE, STRIPE)],
                            sp_h.at[cid, pl.ds(sid * STRIPE, STRIPE)])
        else:
            pltpu.sync_copy(sacc.at[pl.ds(sid * STRIPE, STRIPE)],
                            sp_h.at[cid, pl.ds(sid * STRIPE, STRIPE)])

    return k


def _make_agg(E_pad, N_pad, F, heads, hd_stride, RPS, SCAN):
    MC = 32
    SLAB = E_pad // NS
    NPASS = N_pad // (NC * RPS)
    STRIPE = RPS // NS
    heads16 = heads > 1
    exb_t = pltpu.VMEM((MC, 16) if heads16 else (MC,), jnp.float32)

    @functools.partial(
        pl.kernel,
        out_type=jax.ShapeDtypeStruct((N_pad, F), jnp.float32),
        mesh=_mesh(),
        compiler_params=pltpu.CompilerParams(
            use_tc_tiling_on_sc=False, needs_layout_passes=False),
        scratch_types=[
            pltpu.VMEM((SCAN,), jnp.int32),
            pltpu.VMEM((SCAN,), jnp.int32),
            pltpu.VMEM((SCAN + 96,), jnp.int32),
            pltpu.VMEM((MC,), jnp.int32), pltpu.VMEM((MC,), jnp.int32),
            pltpu.VMEM((MC,), jnp.int32), pltpu.VMEM((MC,), jnp.int32),
            pltpu.VMEM((MC,), jnp.int32), pltpu.VMEM((MC,), jnp.int32),
            exb_t, exb_t,
            pltpu.VMEM((MC, F), jnp.float32),
            pltpu.VMEM((MC, F), jnp.float32),
            pltpu.VMEM((4, F), jnp.float32),
            pltpu.VMEM_SHARED((RPS + 8, F), jnp.float32),
            pltpu.SemaphoreType.DMA, pltpu.SemaphoreType.DMA,
            pltpu.SemaphoreType.DMA, pltpu.SemaphoreType.DMA,
        ],
    )
    def k(src_h, dst_h, ex_h, h_h, out_h,
          dstc, srcc, lst, eidxA, eidxB, srcvA, srcvB, sidxA, sidxB,
          exbA, exbB, hbA, hbB, zbuf, acc, semA1, semA2, semB1, semB2):
        cid = lax.axis_index("c")
        sid = lax.axis_index("s")
        slab0 = sid * SLAB
        iota = _iota16()
        bufs = ((eidxA, srcvA, sidxA, exbA, hbA, semA1, semA2),
                (eidxB, srcvB, sidxB, exbB, hbB, semB1, semB2))
        for r in range(4):
            for j in range(F // 16):
                zbuf[r, pl.ds(j * 16, 16)] = jnp.zeros((16,), jnp.float32)

        def zero_stripe():
            def zrow(b, _):
                pltpu.sync_copy(zbuf, acc.at[pl.ds(sid * STRIPE + b * 4, 4)])
                return 0
            lax.fori_loop(0, STRIPE // 4, zrow, 0)

        zero_stripe()

        @pl.when(sid == 0)
        def _():
            pltpu.sync_copy(zbuf, acc.at[pl.ds(RPS, 4)])
            pltpu.sync_copy(zbuf, acc.at[pl.ds(RPS + 4, 4)])

        plsc.subcore_barrier()

        def fire(q, t, bset):
            eidx, srcv, sidx, exb, hb, s1, s2 = bset
            qb = q * MC
            for kk in range(MC // 16):
                w = lst[pl.ds(qb + kk * 16, 16)]
                lid = jnp.minimum(w & 8191, SCAN - 1)
                eidx[pl.ds(kk * 16, 16)] = lid + (slab0 + t * SCAN)
                sidx[pl.ds(kk * 16, 16)] = jnp.minimum(
                    lax.shift_right_logical(w, 13), RPS)
                srcv[pl.ds(kk * 16, 16)] = plsc.load_gather(srcc, [lid])
            pltpu.async_copy(ex_h.at[eidx], exb, s1)
            pltpu.async_copy(h_h.at[srcv], hb, s2)

        def wait(bset):
            eidx, srcv, sidx, exb, hb, s1, s2 = bset
            pltpu.make_async_copy(ex_h.at[eidx], exb, s1).wait()
            pltpu.make_async_copy(h_h.at[srcv], hb, s2).wait()

        def process(bset):
            eidx, srcv, sidx, exb, hb, s1, s2 = bset

            def edge(i, _3):
                if heads16:
                    for hd in range(heads):
                        wv = plsc.load_gather(
                            exb, [jnp.full((16,), i, jnp.int32),
                                  jnp.full((16,), hd, jnp.int32)])
                        for j in range(hd_stride // 16):
                            col = hd * hd_stride + j * 16
                            hb[i, pl.ds(col, 16)] = hb[i, pl.ds(col, 16)] * wv
                else:
                    wv = plsc.load_gather(exb, [jnp.full((16,), i, jnp.int32)])
                    for j in range(F // 16):
                        hb[i, pl.ds(j * 16, 16)] = hb[i, pl.ds(j * 16, 16)] * wv
                return 0

            lax.fori_loop(0, MC, edge, 0)
            pltpu.sync_copy(hb, acc.at[sidx], add=True)

        def do_pass(p, _):
            lo = p * (NC * RPS) + cid * RPS
            hi = lo + RPS

            def scan_chunk(t, _2):
                pltpu.sync_copy(dst_h.at[pl.ds(slab0 + t * SCAN, SCAN)], dstc)
                pltpu.sync_copy(src_h.at[pl.ds(slab0 + t * SCAN, SCAN)], srcc)

                def vec(kk, cur2):
                    d = dstc[pl.ds(kk * 16, 16)]
                    m = (d >= lo) & (d < hi)
                    lid = iota + kk * 16
                    w = lid | lax.shift_left(d - lo, 13)
                    c = lax.cumsum(m.astype(jnp.int32))
                    pos = jnp.where(m, cur2 + c - 1,
                                    jnp.full((16,), SCAN + 80, jnp.int32))
                    plsc.store_scatter(lst, [pos], w)
                    return cur2 + jnp.max(c)

                cnt = lax.fori_loop(0, SCAN // 16, vec, 0)
                padw = iota | lax.shift_left(RPS + (iota & 7), 13)
                for j in range(5):
                    lst[pl.ds(cnt + j * 16, 16)] = padw
                nq2 = (cnt + 2 * MC - 1) // (2 * MC)
                fire(0, t, bufs[0])

                def pair(j2, _3):
                    fire(2 * j2 + 1, t, bufs[1])
                    wait(bufs[0])
                    process(bufs[0])
                    fire(2 * j2 + 2, t, bufs[0])
                    wait(bufs[1])
                    process(bufs[1])
                    return 0

                lax.fori_loop(0, nq2, pair, 0)
                wait(bufs[0])
                return 0

            lax.fori_loop(0, SLAB // SCAN, scan_chunk, 0)
            plsc.subcore_barrier()
            row0 = p * (NC * RPS) + cid * RPS + sid * STRIPE
            pltpu.sync_copy(acc.at[pl.ds(sid * STRIPE, STRIPE)],
                            out_h.at[pl.ds(row0, STRIPE)])
            zero_stripe()
            plsc.subcore_barrier()
            return 0

        lax.fori_loop(0, NPASS, do_pass, 0)

    return k


def _mm1_body(N, xb_ref, w_ref, as_ref, ad_ref, h_ref, at_ref, dt_ref):
    blk = xb_ref.shape[0]
    h = jnp.dot(xb_ref[...], w_ref[...], preferred_element_type=jnp.float32)
    h3 = h.reshape(blk, 10, 78)
    a_s = jnp.sum(h3 * as_ref[...][None], axis=-1)
    a_d = jnp.sum(h3 * ad_ref[...][None], axis=-1)
    a_s = jnp.concatenate([a_s, jnp.zeros((blk, 6), jnp.float32)], axis=1)
    a_d = jnp.concatenate([a_d, jnp.full((blk, 6), -1e30, jnp.float32)], axis=1)
    rows = pl.program_id(0) * blk + lax.broadcasted_iota(jnp.int32, (blk, 1), 0)
    a_d = jnp.where(rows < N, a_d, -1e30)
    h_ref[...] = jnp.pad(h3, ((0, 0), (0, 0), (0, 2))).reshape(blk, 800)
    at_ref[...] = a_s
    dt_ref[...] = a_d


def _mm1(x_pad, W1, as1, ad1, N):
    N_pad = x_pad.shape[0]
    blk = 512
    grid = N_pad // blk
    return pl.pallas_call(
        functools.partial(_mm1_body, N),
        grid=(grid,),
        in_specs=[
            pl.BlockSpec((blk, 78), lambda i: (i, 0)),
            pl.BlockSpec((78, 780), lambda i: (0, 0)),
            pl.BlockSpec((10, 78), lambda i: (0, 0)),
            pl.BlockSpec((10, 78), lambda i: (0, 0)),
        ],
        out_specs=[
            pl.BlockSpec((blk, 800), lambda i: (i, 0)),
            pl.BlockSpec((blk, 16), lambda i: (i, 0)),
            pl.BlockSpec((blk, 16), lambda i: (i, 0)),
        ],
        out_shape=[
            jax.ShapeDtypeStruct((N_pad, 800), jnp.float32),
            jax.ShapeDtypeStruct((N_pad, 16), jnp.float32),
            jax.ShapeDtypeStruct((N_pad, 16), jnp.float32),
        ],
    )(x_pad, W1, as1, ad1)


def _mm2_body(N, xb_ref, w_ref, as_ref, ad_ref, h_ref, at_ref, dt_ref):
    blk = xb_ref.shape[0]
    h = jnp.dot(xb_ref[...], w_ref[...], preferred_element_type=jnp.float32)
    a_s = jnp.sum(h * as_ref[...], axis=-1, keepdims=True)
    a_d = jnp.sum(h * ad_ref[...], axis=-1, keepdims=True)
    rows = pl.program_id(0) * blk + lax.broadcasted_iota(jnp.int32, (blk, 1), 0)
    a_d = jnp.where(rows < N, a_d, -1e30)
    h_ref[...] = h
    at_ref[...] = a_s
    dt_ref[...] = a_d


def _mm2(h1_pad, W2, as2, ad2, N):
    N_pad = h1_pad.shape[0]
    blk = 512
    grid = N_pad // blk
    return pl.pallas_call(
        functools.partial(_mm2_body, N),
        grid=(grid,),
        in_specs=[
            pl.BlockSpec((blk, 780), lambda i: (i, 0)),
            pl.BlockSpec((780, 128), lambda i: (0, 0)),
            pl.BlockSpec((1, 128), lambda i: (0, 0)),
            pl.BlockSpec((1, 128), lambda i: (0, 0)),
        ],
        out_specs=[
            pl.BlockSpec((blk, 128), lambda i: (i, 0)),
            pl.BlockSpec((blk, 1), lambda i: (i, 0)),
            pl.BlockSpec((blk, 1), lambda i: (i, 0)),
        ],
        out_shape=[
            jax.ShapeDtypeStruct((N_pad, 128), jnp.float32),
            jax.ShapeDtypeStruct((N_pad, 1), jnp.float32),
            jax.ShapeDtypeStruct((N_pad, 1), jnp.float32),
        ],
    )(h1_pad, W2, as2, ad2)


def _head_body(xc_ref, w1_ref, b1_ref, w2_ref, b2_ref, wo_ref, bo_ref, out_ref):
    h1 = jnp.maximum(
        jnp.dot(xc_ref[...], w1_ref[...], preferred_element_type=jnp.float32)
        + b1_ref[...], 0.0)
    h2 = jnp.maximum(
        jnp.dot(h1, w2_ref[...], preferred_element_type=jnp.float32)
        + b2_ref[...], 0.0)
    out_ref[...] = (
        jnp.dot(h2, wo_ref[...], preferred_element_type=jnp.float32) + bo_ref[...])


def _head(xc, W_fc1, b_fc1, W_fc2, b_fc2, W_out, b_out):
    B = xc.shape[0]
    return pl.pallas_call(
        _head_body,
        out_shape=jax.ShapeDtypeStruct((B, 1), jnp.float32),
    )(xc, W_fc1, b_fc1.reshape(1, -1), W_fc2, b_fc2.reshape(1, -1),
      W_out, b_out.reshape(1, -1))


def kernel(x, edge_index, batch, target, W1, as1, ad1, b1, W2, as2, ad2, b2,
           Wg, bg, emb, cw, cb, Wxt, bxt, W_fc1, b_fc1, W_fc2, b_fc2,
           W_out, b_out):
    N = x.shape[0]
    E = edge_index.shape[1]
    B = target.shape[0]
    N_pad = 53248
    E_tot = E + N
    SLAB32 = -(-E_tot // (NW * 2048)) * 2048
    E_pad = NW * SLAB32

    loop = jnp.arange(N, dtype=edge_index.dtype)
    src = jnp.concatenate([edge_index[0], loop])
    dst = jnp.concatenate([edge_index[1], loop])
    npe = E_pad - E_tot
    src_p = jnp.concatenate([src, (jnp.arange(npe, dtype=jnp.int32) * 37) % N])
    dst_p = jnp.concatenate([dst, jnp.full((npe,), N, jnp.int32)])

    x_pad = jnp.pad(x, ((0, N_pad - N), (0, 0)))
    h1p, as1t, ad1t = _mm1(x_pad, W1, as1, ad1, N)

    eph1 = _make_ephase(E_pad, N_pad, True)
    ex1, s1p = eph1(src_p, dst_p, as1t, ad1t)
    agg1 = _make_agg(E_pad, N_pad, 800, 10, 80, 1024, 6656)
    outr1 = agg1(src_p, dst_p, ex1, h1p)

    s1 = (s1p[0] + s1p[1])[:N, :10]
    h1 = outr1.reshape(N_pad, 10, 80)[:N, :, :78] / (s1[:, :, None] + 1e-16)
    h1 = jax.nn.elu(h1.reshape(N, 780) + b1)

    h1a = jnp.pad(h1, ((0, N_pad - N), (0, 0)))
    h2p, as2t, ad2t = _mm2(h1a, W2, as2, ad2, N)
    eph2 = _make_ephase(E_pad, N_pad, False)
    ex2, s2p = eph2(src_p, dst_p, as2t.reshape(-1), ad2t.reshape(-1))
    agg2 = _make_agg(E_pad, N_pad, 128, 1, 128, 13312, 4096)
    outr2 = agg2(src_p, dst_p, ex2, h2p)

    s2 = (s2p[0] + s2p[1])[:N]
    h2 = outr2[:N] / (s2[:, None] + 1e-16) + b2
    h = jax.nn.relu(h2)

    g = jax.ops.segment_max(h, batch, num_segments=B)
    g = jax.nn.relu(g @ Wg + bg)
    emb_t = jnp.take(emb, target, axis=0)
    conv = jax.lax.conv_general_dilated(
        emb_t, cw, (1,), 'VALID', dimension_numbers=('NCH', 'OIH', 'NCH'))
    conv = jax.nn.relu(conv + cb[None, :, None])
    xt = conv.reshape(B, 32 * 121) @ Wxt + bxt
    xc = jnp.concatenate([g, xt], axis=1)
    return _head(xc, W_fc1, b_fc1, W_fc2, b_fc2, W_out, b_out)
